# baseline (speedup 1.0000x reference)
import jax
import jax.numpy as jnp
from jax.experimental import pallas as pl


def _shift_kernel(x_ref, sel_ref, o_ref):
    HW = 196
    xv = x_ref[...]
    o_ref[0] = xv
    shifted = jnp.concatenate([xv[:, :, HW:], xv[:, :, -HW:]], axis=2)
    o_ref[1] = jnp.where(sel_ref[...] != 0, xv, shifted)


def kernel(x, t_length):
    N, C, T, H, W = x.shape
    HW = H * W
    L = T * HW
    Cb = 128
    idx = jnp.mod(t_length.astype(jnp.int32) - 1, T)
    mask = jnp.zeros((T,), jnp.float32).at[idx].set(1.0)
    sel = jnp.repeat(mask, HW).reshape(1, 1, L)
    xr = x.reshape(N, C, L)

    out = pl.pallas_call(
        _shift_kernel,
        grid=(N, C // Cb),
        in_specs=[
            pl.BlockSpec((1, Cb, L), lambda n, c: (n, c, 0)),
            pl.BlockSpec((1, 1, L), lambda n, c: (0, 0, 0)),
        ],
        out_specs=pl.BlockSpec((2, 1, Cb, L), lambda n, c: (0, n, c, 0)),
        out_shape=jax.ShapeDtypeStruct((2, N, C, L), x.dtype),
    )(xr, sel)
    return out.reshape(2, N, C, T, H, W)


# speedup vs baseline: 1.0019x; 1.0019x over previous
import jax
import jax.numpy as jnp
from jax.experimental import pallas as pl
from jax.experimental.pallas import tpu as pltpu


def _shift_kernel(x_ref, sel_ref, o_ref):
    HW = 196
    xv = x_ref[...]
    o_ref[0] = xv
    shifted = jnp.concatenate([xv[:, :, HW:], xv[:, :, -HW:]], axis=2)
    o_ref[1] = jnp.where(sel_ref[...] != 0, xv, shifted)


def kernel(x, t_length):
    N, C, T, H, W = x.shape
    HW = H * W
    L = T * HW
    Cb = 128
    idx = jnp.mod(t_length.astype(jnp.int32) - 1, T)
    mask = jnp.zeros((T,), jnp.float32).at[idx].set(1.0)
    sel = jnp.repeat(mask, HW).reshape(1, 1, L)
    xr = x.reshape(N, C, L)

    out = pl.pallas_call(
        _shift_kernel,
        grid=(N, C // Cb),
        in_specs=[
            pl.BlockSpec((1, Cb, L), lambda n, c: (n, c, 0)),
            pl.BlockSpec((1, 1, L), lambda n, c: (0, 0, 0)),
        ],
        out_specs=pl.BlockSpec((2, 1, Cb, L), lambda n, c: (0, n, c, 0)),
        out_shape=jax.ShapeDtypeStruct((2, N, C, L), x.dtype),
        compiler_params=pltpu.CompilerParams(
            dimension_semantics=("parallel", "parallel")
        ),
    )(xr, sel)
    return out.reshape(2, N, C, T, H, W)
